# baseline (device time: 131921 ns/iter reference)
import jax
import jax.numpy as jnp
from jax import lax
from jax.experimental import pallas as pl
from jax.experimental.pallas import tpu as pltpu

N_DEV = 16
T = 1024
D = 512
H = 1024
E = 64
E_LOC = 4
CAP = 204
CAP_PAD = 208

C_SEND, C_RECV, X_SEND, X_RECV, M_SEND, M_RECV, Y_SEND, Y_RECV = range(8)
LOGICAL = pl.DeviceIdType.LOGICAL
f32 = jnp.float32
i32 = jnp.int32


def kernel(x, router_W, route_idx, expert_W):
    del router_W

    def body(x_ref, route_ref, w_ref, out_ref,
             counts_mine, counts_vmem, stage_vmem, tbl_vmem, meta_src,
             xbuf, ybuf, meta_vmem, stage_smem, tok_smem, meta_smem,
             sems, local_sem, exit_sem):
        my = lax.axis_index("i")

        with jax.named_scope("prep"):
            out_ref[...] = jnp.zeros((T, H), f32)
            rt = route_ref[...]
            eids = lax.broadcasted_iota(i32, (T, E), 1)
            oh = (rt == eids).astype(f32)
            counts_mine[...] = jnp.sum(oh, axis=0, keepdims=True).astype(i32)
            meta_src[...] = my * T + lax.broadcasted_iota(i32, (T, 1), 0)

            lt_tok = (lax.broadcasted_iota(i32, (T, T), 1)
                      < lax.broadcasted_iota(i32, (T, T), 0)).astype(f32)
            cum = jnp.dot(lt_tok, oh, preferred_element_type=f32)
            localrank = jnp.sum(cum * oh, axis=1, keepdims=True)

        barrier_sem = pltpu.get_barrier_semaphore()

        with jax.named_scope("barrier"):
            def bsig(d, c):
                @pl.when(d != my)
                def _():
                    pl.semaphore_signal(barrier_sem, 1, device_id=d,
                                        device_id_type=LOGICAL)
                return c
            lax.fori_loop(0, N_DEV, bsig, 0)
            pl.semaphore_wait(barrier_sem, N_DEV - 1)

        def loop_wait(n, f):
            def b(i, c):
                f()
                return c
            lax.fori_loop(0, n, b, 0)

        with jax.named_scope("counts_ag"):
            def csend(d, c):
                pltpu.make_async_remote_copy(
                    counts_mine, counts_vmem.at[pl.ds(my, 1), :],
                    sems.at[C_SEND], sems.at[C_RECV],
                    device_id=d, device_id_type=LOGICAL,
                ).start()
                return c
            lax.fori_loop(0, N_DEV, csend, 0)

            c_wait = pltpu.make_async_remote_copy(
                counts_mine, counts_vmem.at[pl.ds(0, 1), :],
                sems.at[C_SEND], sems.at[C_RECV],
                device_id=my, device_id_type=LOGICAL)

            loop_wait(N_DEV, c_wait.wait_recv)
            loop_wait(N_DEV, c_wait.wait_send)

        with jax.named_scope("offsets"):
            cv = counts_vmem[...]
            rowid = lax.broadcasted_iota(i32, (N_DEV, E), 0)
            offs = jnp.sum(jnp.where(rowid < my, cv, 0), axis=0)
            tots = jnp.sum(cv, axis=0)

            offvec = jnp.sum(oh * offs[None, :].astype(f32), axis=1,
                             keepdims=True)
            rank = offvec + localrank
            keep_vec = rank < float(CAP)
            row_vec = (rt % E_LOC) * CAP_PAD + rank.astype(i32)
            tbl_vmem[...] = jnp.concatenate(
                [keep_vec.astype(i32), rt // E_LOC, row_vec], axis=1)

            kept_mine_s = jnp.sum(keep_vec.astype(i32))
            stage_vmem[...] = jnp.concatenate(
                [offs[None, :], tots[None, :],
                 jnp.zeros((1, E), i32) + kept_mine_s], 0)

            scopy = pltpu.make_async_copy(stage_vmem, stage_smem, local_sem)
            scopy.start()
            scopy.wait()
            kcopy = pltpu.make_async_copy(tbl_vmem, tok_smem, local_sem)
            kcopy.start()
            kcopy.wait()

        kept_mine = stage_smem[2, 0]

        def dis(i, c):
            @pl.when(tok_smem[i, 0] == 1)
            def _():
                dev = tok_smem[i, 1]
                row = tok_smem[i, 2]
                pltpu.make_async_remote_copy(
                    x_ref.at[pl.ds(i, 1), :], xbuf.at[pl.ds(row, 1), :],
                    sems.at[X_SEND], sems.at[X_RECV],
                    device_id=dev, device_id_type=LOGICAL,
                ).start()
                pltpu.make_async_remote_copy(
                    meta_src.at[pl.ds(i, 1), :], meta_vmem.at[pl.ds(row, 1), :],
                    sems.at[M_SEND], sems.at[M_RECV],
                    device_id=dev, device_id_type=LOGICAL,
                ).start()
            return c

        with jax.named_scope("dispatch"):
            lax.fori_loop(0, T, dis, 0, unroll=8)

        kept_slot = [jnp.minimum(stage_smem[1, my * E_LOC + s], CAP)
                     for s in range(E_LOC)]
        kept_owner = kept_slot[0] + kept_slot[1] + kept_slot[2] + kept_slot[3]

        x_wait = pltpu.make_async_remote_copy(
            x_ref.at[pl.ds(0, 1), :], xbuf.at[pl.ds(0, 1), :],
            sems.at[X_SEND], sems.at[X_RECV],
            device_id=my, device_id_type=LOGICAL)
        m_wait = pltpu.make_async_remote_copy(
            meta_src.at[pl.ds(0, 1), :], meta_vmem.at[pl.ds(0, 1), :],
            sems.at[M_SEND], sems.at[M_RECV],
            device_id=my, device_id_type=LOGICAL)
        y_wait = pltpu.make_async_remote_copy(
            ybuf.at[pl.ds(0, 1), :], out_ref.at[pl.ds(0, 1), :],
            sems.at[Y_SEND], sems.at[Y_RECV],
            device_id=my, device_id_type=LOGICAL)

        with jax.named_scope("wait_x"):
            loop_wait(kept_owner, x_wait.wait_recv)
            loop_wait(kept_owner, m_wait.wait_recv)

        with jax.named_scope("gemm"):
            for s in range(E_LOC):
                a = xbuf[s * CAP_PAD:(s + 1) * CAP_PAD, :]
                ybuf[s * CAP_PAD:(s + 1) * CAP_PAD, :] = jnp.dot(
                    a, w_ref[s], preferred_element_type=f32)

            mcopy = pltpu.make_async_copy(meta_vmem, meta_smem, local_sem)
            mcopy.start()
            mcopy.wait()

        with jax.named_scope("combine"):
            for s in range(E_LOC):
                base = s * CAP_PAD
                ks = kept_slot[s]

                def comb(rr, c, base=base, ks=ks):
                    @pl.when(rr < ks)
                    def _():
                        m = meta_smem[base + rr, 0]
                        pltpu.make_async_remote_copy(
                            ybuf.at[pl.ds(base + rr, 1), :],
                            out_ref.at[pl.ds(m % T, 1), :],
                            sems.at[Y_SEND], sems.at[Y_RECV],
                            device_id=m // T, device_id_type=LOGICAL,
                        ).start()
                    return c
                lax.fori_loop(0, CAP, comb, 0, unroll=8)

        with jax.named_scope("drain"):
            loop_wait(kept_mine, x_wait.wait_send)
            loop_wait(kept_mine, m_wait.wait_send)
            loop_wait(kept_owner, y_wait.wait_send)
            loop_wait(kept_mine, y_wait.wait_recv)

        with jax.named_scope("exit_barrier"):
            def esig(d, c):
                @pl.when(d != my)
                def _():
                    pl.semaphore_signal(exit_sem, 1, device_id=d,
                                        device_id_type=LOGICAL)
                return c
            lax.fori_loop(0, N_DEV, esig, 0)
            pl.semaphore_wait(exit_sem, N_DEV - 1)

    return pl.pallas_call(
        body,
        out_shape=jax.ShapeDtypeStruct((T, H), f32),
        in_specs=[pl.BlockSpec(memory_space=pltpu.VMEM)] * 3,
        out_specs=pl.BlockSpec(memory_space=pltpu.VMEM),
        scratch_shapes=[
            pltpu.VMEM((1, E), i32),
            pltpu.VMEM((N_DEV, E), i32),
            pltpu.VMEM((3, E), i32),
            pltpu.VMEM((T, 3), i32),
            pltpu.VMEM((T, 1), i32),
            pltpu.VMEM((E_LOC * CAP_PAD, D), f32),
            pltpu.VMEM((E_LOC * CAP_PAD, H), f32),
            pltpu.VMEM((E_LOC * CAP_PAD, 1), i32),
            pltpu.SMEM((3, E), i32),
            pltpu.SMEM((T, 3), i32),
            pltpu.SMEM((E_LOC * CAP_PAD, 1), i32),
            pltpu.SemaphoreType.DMA((8,)),
            pltpu.SemaphoreType.DMA,
            pltpu.SemaphoreType.REGULAR,
        ],
        compiler_params=pltpu.CompilerParams(collective_id=0),
    )(x, route_idx, expert_W)


# device time: 124261 ns/iter; 1.0616x vs baseline; 1.0616x over previous
import jax
import jax.numpy as jnp
from jax import lax
from jax.experimental import pallas as pl
from jax.experimental.pallas import tpu as pltpu

N_DEV = 16
T = 1024
D = 512
DX = 640
H = 1024
E = 64
E_LOC = 4
CAP = 204
CAP_PAD = 208

C_SEND, C_RECV, X_SEND, X_RECV, M_SEND, M_RECV, Y_SEND, Y_RECV = range(8)
LOGICAL = pl.DeviceIdType.LOGICAL
f32 = jnp.float32
i32 = jnp.int32


def kernel(x, router_W, route_idx, expert_W):
    del router_W

    def body(x_ref, route_ref, w_ref, out_ref,
             counts_mine, counts_vmem, stage_vmem, tbl_vmem, xext,
             xbuf, ybuf, meta_vmem, stage_smem, tok_smem, meta_smem,
             sems, local_sem, exit_sem):
        my = lax.axis_index("i")

        with jax.named_scope("prep"):
            out_ref[...] = jnp.zeros((T, H), f32)
            rt = route_ref[...]
            eids = lax.broadcasted_iota(i32, (T, E), 1)
            oh = (rt == eids).astype(f32)
            counts_mine[...] = jnp.sum(oh, axis=0, keepdims=True).astype(i32)
            metaf = (my * T + lax.broadcasted_iota(i32, (T, 1), 0)).astype(f32)
            xext[...] = jnp.concatenate(
                [x_ref[...], metaf, jnp.zeros((T, DX - D - 1), f32)], axis=1)

            lt_tok = (lax.broadcasted_iota(i32, (T, T), 1)
                      < lax.broadcasted_iota(i32, (T, T), 0)).astype(f32)
            cum = jnp.dot(lt_tok, oh, preferred_element_type=f32)
            localrank = jnp.sum(cum * oh, axis=1, keepdims=True)

        barrier_sem = pltpu.get_barrier_semaphore()

        with jax.named_scope("barrier"):
            def bsig(d, c):
                @pl.when(d != my)
                def _():
                    pl.semaphore_signal(barrier_sem, 1, device_id=d,
                                        device_id_type=LOGICAL)
                return c
            lax.fori_loop(0, N_DEV, bsig, 0)
            pl.semaphore_wait(barrier_sem, N_DEV - 1)

        def loop_wait(n, f):
            def b(i, c):
                f()
                return c
            lax.fori_loop(0, n, b, 0)

        with jax.named_scope("counts_ag"):
            def csend(d, c):
                pltpu.make_async_remote_copy(
                    counts_mine, counts_vmem.at[pl.ds(my, 1), :],
                    sems.at[C_SEND], sems.at[C_RECV],
                    device_id=d, device_id_type=LOGICAL,
                ).start()
                return c
            lax.fori_loop(0, N_DEV, csend, 0)

            c_wait = pltpu.make_async_remote_copy(
                counts_mine, counts_vmem.at[pl.ds(0, 1), :],
                sems.at[C_SEND], sems.at[C_RECV],
                device_id=my, device_id_type=LOGICAL)

            loop_wait(N_DEV, c_wait.wait_recv)
            loop_wait(N_DEV, c_wait.wait_send)

        with jax.named_scope("offsets"):
            cv = counts_vmem[...]
            rowid = lax.broadcasted_iota(i32, (N_DEV, E), 0)
            offs = jnp.sum(jnp.where(rowid < my, cv, 0), axis=0)
            tots = jnp.sum(cv, axis=0)

            offvec = jnp.sum(oh * offs[None, :].astype(f32), axis=1,
                             keepdims=True)
            rank = offvec + localrank
            keep_vec = rank < float(CAP)
            row_vec = (rt % E_LOC) * CAP_PAD + rank.astype(i32)
            tbl_vmem[...] = jnp.concatenate(
                [keep_vec.astype(i32), rt // E_LOC, row_vec], axis=1)

            kept_mine_s = jnp.sum(keep_vec.astype(i32))
            stage_vmem[...] = jnp.concatenate(
                [offs[None, :], tots[None, :],
                 jnp.zeros((1, E), i32) + kept_mine_s], 0)

            scopy = pltpu.make_async_copy(stage_vmem, stage_smem, local_sem)
            scopy.start()
            scopy.wait()
            kcopy = pltpu.make_async_copy(tbl_vmem, tok_smem, local_sem)
            kcopy.start()
            kcopy.wait()

        kept_mine = stage_smem[2, 0]

        def dis(i, c):
            @pl.when(tok_smem[i, 0] == 1)
            def _():
                dev = tok_smem[i, 1]
                row = tok_smem[i, 2]
                pltpu.make_async_remote_copy(
                    xext.at[pl.ds(i, 1), :], xbuf.at[pl.ds(row, 1), :],
                    sems.at[X_SEND], sems.at[X_RECV],
                    device_id=dev, device_id_type=LOGICAL,
                ).start()
            return c

        with jax.named_scope("dispatch"):
            lax.fori_loop(0, T, dis, 0, unroll=8)

        kept_slot = [jnp.minimum(stage_smem[1, my * E_LOC + s], CAP)
                     for s in range(E_LOC)]
        kept_owner = kept_slot[0] + kept_slot[1] + kept_slot[2] + kept_slot[3]

        x_wait = pltpu.make_async_remote_copy(
            xext.at[pl.ds(0, 1), :], xbuf.at[pl.ds(0, 1), :],
            sems.at[X_SEND], sems.at[X_RECV],
            device_id=my, device_id_type=LOGICAL)
        y_wait = pltpu.make_async_remote_copy(
            ybuf.at[pl.ds(0, 1), :], out_ref.at[pl.ds(0, 1), :],
            sems.at[Y_SEND], sems.at[Y_RECV],
            device_id=my, device_id_type=LOGICAL)

        with jax.named_scope("wait_x"):
            loop_wait(kept_owner, x_wait.wait_recv)

        with jax.named_scope("gemm"):
            meta_vmem[...] = xbuf[:, D:D + 1].astype(i32)
            mcopy = pltpu.make_async_copy(meta_vmem, meta_smem, local_sem)
            mcopy.start()
            for s in range(E_LOC):
                a = xbuf[s * CAP_PAD:(s + 1) * CAP_PAD, 0:D]
                ybuf[s * CAP_PAD:(s + 1) * CAP_PAD, :] = jnp.dot(
                    a, w_ref[s], preferred_element_type=f32)
            mcopy.wait()

        with jax.named_scope("combine"):
            for s in range(E_LOC):
                base = s * CAP_PAD
                ks = kept_slot[s]

                def comb(rr, c, base=base, ks=ks):
                    @pl.when(rr < ks)
                    def _():
                        m = meta_smem[base + rr, 0]
                        pltpu.make_async_remote_copy(
                            ybuf.at[pl.ds(base + rr, 1), :],
                            out_ref.at[pl.ds(m % T, 1), :],
                            sems.at[Y_SEND], sems.at[Y_RECV],
                            device_id=m // T, device_id_type=LOGICAL,
                        ).start()
                    return c
                lax.fori_loop(0, CAP, comb, 0, unroll=8)

        with jax.named_scope("drain"):
            loop_wait(kept_mine, x_wait.wait_send)
            loop_wait(kept_owner, y_wait.wait_send)
            loop_wait(kept_mine, y_wait.wait_recv)

        with jax.named_scope("exit_barrier"):
            def esig(d, c):
                @pl.when(d != my)
                def _():
                    pl.semaphore_signal(exit_sem, 1, device_id=d,
                                        device_id_type=LOGICAL)
                return c
            lax.fori_loop(0, N_DEV, esig, 0)
            pl.semaphore_wait(exit_sem, N_DEV - 1)

    return pl.pallas_call(
        body,
        out_shape=jax.ShapeDtypeStruct((T, H), f32),
        in_specs=[pl.BlockSpec(memory_space=pltpu.VMEM)] * 3,
        out_specs=pl.BlockSpec(memory_space=pltpu.VMEM),
        scratch_shapes=[
            pltpu.VMEM((1, E), i32),
            pltpu.VMEM((N_DEV, E), i32),
            pltpu.VMEM((3, E), i32),
            pltpu.VMEM((T, 3), i32),
            pltpu.VMEM((T, DX), f32),
            pltpu.VMEM((E_LOC * CAP_PAD, DX), f32),
            pltpu.VMEM((E_LOC * CAP_PAD, H), f32),
            pltpu.VMEM((E_LOC * CAP_PAD, 1), i32),
            pltpu.SMEM((3, E), i32),
            pltpu.SMEM((T, 3), i32),
            pltpu.SMEM((E_LOC * CAP_PAD, 1), i32),
            pltpu.SemaphoreType.DMA((8,)),
            pltpu.SemaphoreType.DMA,
            pltpu.SemaphoreType.REGULAR,
        ],
        compiler_params=pltpu.CompilerParams(collective_id=0),
    )(x, route_idx, expert_W)


# device time: 119780 ns/iter; 1.1014x vs baseline; 1.0374x over previous
import jax
import jax.numpy as jnp
from jax import lax
from jax.experimental import pallas as pl
from jax.experimental.pallas import tpu as pltpu

N_DEV = 16
T = 1024
D = 512
DX = 520
H = 1024
E = 64
E_LOC = 4
CAP = 204
CAP_PAD = 208

C_SEND, C_RECV, X_SEND, XR0, XR1, XR2, XR3, Y_SEND, Y_RECV = range(9)
LOGICAL = pl.DeviceIdType.LOGICAL
f32 = jnp.float32
i32 = jnp.int32


def kernel(x, router_W, route_idx, expert_W):
    del router_W

    def body(x_ref, route_ref, w_ref, out_ref,
             counts_mine, counts_vmem, stage_vmem, tbl_vmem, xext,
             xbuf, ybuf, returns, meta_vmem, stage_smem, tok_smem,
             meta_smem, sems, local_sem, exit_sem):
        my = lax.axis_index("i")

        with jax.named_scope("prep"):
            rt = route_ref[...]
            eids = lax.broadcasted_iota(i32, (T, E), 1)
            oh = (rt == eids).astype(f32)
            counts_mine[...] = jnp.sum(oh, axis=0, keepdims=True).astype(i32)
            metaf = (my * T + lax.broadcasted_iota(i32, (T, 1), 0)).astype(f32)
            xext[...] = jnp.concatenate(
                [x_ref[...], metaf, jnp.zeros((T, DX - D - 1), f32)], axis=1)

            lt_tok = (lax.broadcasted_iota(i32, (T, T), 1)
                      < lax.broadcasted_iota(i32, (T, T), 0)).astype(f32)
            cum = jnp.dot(lt_tok, oh, preferred_element_type=f32)
            localrank = jnp.sum(cum * oh, axis=1, keepdims=True)

        barrier_sem = pltpu.get_barrier_semaphore()

        with jax.named_scope("barrier"):
            def bsig(d, c):
                @pl.when(d != my)
                def _():
                    pl.semaphore_signal(barrier_sem, 1, device_id=d,
                                        device_id_type=LOGICAL)
                return c
            lax.fori_loop(0, N_DEV, bsig, 0)
            pl.semaphore_wait(barrier_sem, N_DEV - 1)

        def loop_wait(n, f):
            def b(i, c):
                f()
                return c
            lax.fori_loop(0, n, b, 0)

        with jax.named_scope("counts_ag"):
            def csend(d, c):
                pltpu.make_async_remote_copy(
                    counts_mine, counts_vmem.at[pl.ds(my, 1), :],
                    sems.at[C_SEND], sems.at[C_RECV],
                    device_id=d, device_id_type=LOGICAL,
                ).start()
                return c
            lax.fori_loop(0, N_DEV, csend, 0)

            c_wait = pltpu.make_async_remote_copy(
                counts_mine, counts_vmem.at[pl.ds(0, 1), :],
                sems.at[C_SEND], sems.at[C_RECV],
                device_id=my, device_id_type=LOGICAL)

            loop_wait(N_DEV, c_wait.wait_recv)
            loop_wait(N_DEV, c_wait.wait_send)

        with jax.named_scope("offsets"):
            cv = counts_vmem[...]
            rowid = lax.broadcasted_iota(i32, (N_DEV, E), 0)
            offs = jnp.sum(jnp.where(rowid < my, cv, 0), axis=0)
            tots = jnp.sum(cv, axis=0)

            offvec = jnp.sum(oh * offs[None, :].astype(f32), axis=1,
                             keepdims=True)
            rank = offvec + localrank
            keep_vec = rank < float(CAP)
            slot_vec = rt % E_LOC
            row_vec = slot_vec * CAP_PAD + rank.astype(i32)
            tbl_vmem[...] = jnp.concatenate(
                [keep_vec.astype(i32), rt // E_LOC, row_vec, slot_vec],
                axis=1)

            kept_mine_s = jnp.sum(keep_vec.astype(i32))
            stage_vmem[...] = jnp.concatenate(
                [offs[None, :], tots[None, :],
                 jnp.zeros((1, E), i32) + kept_mine_s], 0)

            scopy = pltpu.make_async_copy(stage_vmem, stage_smem, local_sem)
            scopy.start()
            scopy.wait()
            kcopy = pltpu.make_async_copy(tbl_vmem, tok_smem, local_sem)
            kcopy.start()
            kcopy.wait()

        kept_mine = stage_smem[2, 0]

        def dis(i, c):
            @pl.when(tok_smem[i, 0] == 1)
            def _():
                dev = tok_smem[i, 1]
                row = tok_smem[i, 2]
                slot = tok_smem[i, 3]
                pltpu.make_async_remote_copy(
                    xext.at[pl.ds(i, 1), :], xbuf.at[pl.ds(row, 1), :],
                    sems.at[X_SEND], sems.at[XR0 + slot],
                    device_id=dev, device_id_type=LOGICAL,
                ).start()
            return c

        with jax.named_scope("dispatch"):
            lax.fori_loop(0, T, dis, 0, unroll=8)

        kept_slot = [jnp.minimum(stage_smem[1, my * E_LOC + s], CAP)
                     for s in range(E_LOC)]
        kept_owner = kept_slot[0] + kept_slot[1] + kept_slot[2] + kept_slot[3]

        x_waits = [pltpu.make_async_remote_copy(
            xext.at[pl.ds(0, 1), :], xbuf.at[pl.ds(0, 1), :],
            sems.at[X_SEND], sems.at[XR0 + s],
            device_id=my, device_id_type=LOGICAL) for s in range(E_LOC)]
        y_wait = pltpu.make_async_remote_copy(
            ybuf.at[pl.ds(0, 1), :], returns.at[pl.ds(0, 1), :],
            sems.at[Y_SEND], sems.at[Y_RECV],
            device_id=my, device_id_type=LOGICAL)

        for s in range(E_LOC):
            base = s * CAP_PAD
            ks = kept_slot[s]

            with jax.named_scope(f"slot{s}"):
                loop_wait(ks, x_waits[s].wait_recv)

                meta_vmem[base:base + CAP_PAD, :] = (
                    xbuf[base:base + CAP_PAD, D:D + 1].astype(i32))
                mcopy = pltpu.make_async_copy(
                    meta_vmem.at[pl.ds(base, CAP_PAD), :],
                    meta_smem.at[pl.ds(base, CAP_PAD), :], local_sem)
                mcopy.start()

                a = xbuf[base:base + CAP_PAD, 0:D]
                ybuf[base:base + CAP_PAD, :] = jnp.dot(
                    a, w_ref[s], preferred_element_type=f32)
                mcopy.wait()

                def comb(rr, c, base=base, ks=ks):
                    @pl.when(rr < ks)
                    def _():
                        m = meta_smem[base + rr, 0]
                        pltpu.make_async_remote_copy(
                            ybuf.at[pl.ds(base + rr, 1), :],
                            returns.at[pl.ds(m % T, 1), :],
                            sems.at[Y_SEND], sems.at[Y_RECV],
                            device_id=m // T, device_id_type=LOGICAL,
                        ).start()
                    return c
                lax.fori_loop(0, CAP, comb, 0, unroll=8)

        with jax.named_scope("drain"):
            loop_wait(kept_mine, x_waits[0].wait_send)
            loop_wait(kept_owner, y_wait.wait_send)
            loop_wait(kept_mine, y_wait.wait_recv)

        with jax.named_scope("out_cast"):
            out_ref[...] = jnp.where(keep_vec, returns[...], 0.0)

        with jax.named_scope("exit_barrier"):
            def esig(d, c):
                @pl.when(d != my)
                def _():
                    pl.semaphore_signal(exit_sem, 1, device_id=d,
                                        device_id_type=LOGICAL)
                return c
            lax.fori_loop(0, N_DEV, esig, 0)
            pl.semaphore_wait(exit_sem, N_DEV - 1)

    return pl.pallas_call(
        body,
        out_shape=jax.ShapeDtypeStruct((T, H), f32),
        in_specs=[pl.BlockSpec(memory_space=pltpu.VMEM)] * 3,
        out_specs=pl.BlockSpec(memory_space=pltpu.VMEM),
        scratch_shapes=[
            pltpu.VMEM((1, E), i32),
            pltpu.VMEM((N_DEV, E), i32),
            pltpu.VMEM((3, E), i32),
            pltpu.VMEM((T, 4), i32),
            pltpu.VMEM((T, DX), f32),
            pltpu.VMEM((E_LOC * CAP_PAD, DX), f32),
            pltpu.VMEM((E_LOC * CAP_PAD, H), f32),
            pltpu.VMEM((T, H), f32),
            pltpu.VMEM((E_LOC * CAP_PAD, 1), i32),
            pltpu.SMEM((3, E), i32),
            pltpu.SMEM((T, 4), i32),
            pltpu.SMEM((E_LOC * CAP_PAD, 1), i32),
            pltpu.SemaphoreType.DMA((9,)),
            pltpu.SemaphoreType.DMA,
            pltpu.SemaphoreType.REGULAR,
        ],
        compiler_params=pltpu.CompilerParams(collective_id=0),
    )(x, route_idx, expert_W)


# device time: 116563 ns/iter; 1.1318x vs baseline; 1.0276x over previous
import jax
import jax.numpy as jnp
from jax import lax
from jax.experimental import pallas as pl
from jax.experimental.pallas import tpu as pltpu

N_DEV = 16
T = 1024
D = 512
DX = 520
H = 1024
E = 64
E_LOC = 4
CAP = 204
CAP_PAD = 208

C_SEND, C_RECV, X_SEND, XR0, XR1, XR2, XR3, Y_SEND, Y_RECV = range(9)
LOGICAL = pl.DeviceIdType.LOGICAL
f32 = jnp.float32
i32 = jnp.int32


def kernel(x, router_W, route_idx, expert_W):
    del router_W

    def body(x_ref, route_ref, w_ref, out_ref,
             counts_mine, counts_vmem, stage_vmem, tbl_vmem, xext,
             xbuf, ybuf, returns, meta_vmem, stage_smem, tok_smem,
             meta_smem, sems, local_sem, exit_sem):
        my = lax.axis_index("i")

        with jax.named_scope("prep"):
            rt = route_ref[...]
            eids = lax.broadcasted_iota(i32, (T, E), 1)
            oh = (rt == eids).astype(f32)
            counts_mine[...] = jnp.sum(oh, axis=0, keepdims=True).astype(i32)
            metaf = (my * T + lax.broadcasted_iota(i32, (T, 1), 0)).astype(f32)
            xext[...] = jnp.concatenate(
                [x_ref[...], metaf, jnp.zeros((T, DX - D - 1), f32)], axis=1)

            lt_tok = (lax.broadcasted_iota(i32, (T, T), 1)
                      < lax.broadcasted_iota(i32, (T, T), 0)).astype(f32)
            cum = jnp.dot(lt_tok, oh, preferred_element_type=f32)
            localrank = jnp.sum(cum * oh, axis=1, keepdims=True)

        barrier_sem = pltpu.get_barrier_semaphore()

        with jax.named_scope("barrier"):
            def bsig(d, c):
                @pl.when(d != my)
                def _():
                    pl.semaphore_signal(barrier_sem, 1, device_id=d,
                                        device_id_type=LOGICAL)
                return c
            lax.fori_loop(0, N_DEV, bsig, 0)
            pl.semaphore_wait(barrier_sem, N_DEV - 1)

        def loop_wait(n, f):
            def b(i, c):
                f()
                return c
            lax.fori_loop(0, n, b, 0)

        with jax.named_scope("counts_ag"):
            def csend(d, c):
                pltpu.make_async_remote_copy(
                    counts_mine, counts_vmem.at[pl.ds(my, 1), :],
                    sems.at[C_SEND], sems.at[C_RECV],
                    device_id=d, device_id_type=LOGICAL,
                ).start()
                return c
            lax.fori_loop(0, N_DEV, csend, 0)

            c_wait = pltpu.make_async_remote_copy(
                counts_vmem, counts_vmem,
                sems.at[C_SEND], sems.at[C_RECV],
                device_id=my, device_id_type=LOGICAL)
            c_wait.wait_recv()
            c_wait.wait_send()

        with jax.named_scope("offsets"):
            cv = counts_vmem[...]
            rowid = lax.broadcasted_iota(i32, (N_DEV, E), 0)
            offs = jnp.sum(jnp.where(rowid < my, cv, 0), axis=0)
            tots = jnp.sum(cv, axis=0)

            offvec = jnp.sum(oh * offs[None, :].astype(f32), axis=1,
                             keepdims=True)
            rank = offvec + localrank
            keep_vec = rank < float(CAP)
            slot_vec = rt % E_LOC
            row_vec = slot_vec * CAP_PAD + rank.astype(i32)
            tbl_vmem[...] = jnp.concatenate(
                [keep_vec.astype(i32), rt // E_LOC, row_vec, slot_vec],
                axis=1)

            kept_mine_s = jnp.sum(keep_vec.astype(i32))
            stage_vmem[...] = jnp.concatenate(
                [offs[None, :], tots[None, :],
                 jnp.zeros((1, E), i32) + kept_mine_s], 0)

            scopy = pltpu.make_async_copy(stage_vmem, stage_smem, local_sem)
            scopy.start()
            scopy.wait()
            kcopy = pltpu.make_async_copy(tbl_vmem, tok_smem, local_sem)
            kcopy.start()
            kcopy.wait()

        kept_mine = stage_smem[2, 0]

        def dis(i, c):
            @pl.when(tok_smem[i, 0] == 1)
            def _():
                dev = tok_smem[i, 1]
                row = tok_smem[i, 2]
                slot = tok_smem[i, 3]
                pltpu.make_async_remote_copy(
                    xext.at[pl.ds(i, 1), :], xbuf.at[pl.ds(row, 1), :],
                    sems.at[X_SEND], sems.at[XR0 + slot],
                    device_id=dev, device_id_type=LOGICAL,
                ).start()
            return c

        with jax.named_scope("dispatch"):
            lax.fori_loop(0, T, dis, 0, unroll=8)

        kept_slot = [jnp.minimum(stage_smem[1, my * E_LOC + s], CAP)
                     for s in range(E_LOC)]
        kept_owner = kept_slot[0] + kept_slot[1] + kept_slot[2] + kept_slot[3]

        SZ8 = (128, 64, 32, 16, 8, 4, 2, 1)
        SZ11 = (1024, 512, 256, 128, 64, 32, 16, 8, 4, 2, 1)
        xr_waits = {(s, sz): pltpu.make_async_remote_copy(
            xbuf.at[pl.ds(0, sz), :], xbuf.at[pl.ds(0, sz), :],
            sems.at[X_SEND], sems.at[XR0 + s],
            device_id=my, device_id_type=LOGICAL)
            for s in range(E_LOC) for sz in SZ8}
        xs_waits = {sz: pltpu.make_async_remote_copy(
            xext.at[pl.ds(0, sz), :], xext.at[pl.ds(0, sz), :],
            sems.at[X_SEND], sems.at[XR0],
            device_id=my, device_id_type=LOGICAL) for sz in SZ11}
        ys_waits = {sz: pltpu.make_async_remote_copy(
            ybuf.at[pl.ds(0, sz), :], ybuf.at[pl.ds(0, sz), :],
            sems.at[Y_SEND], sems.at[Y_RECV],
            device_id=my, device_id_type=LOGICAL) for sz in SZ11 if sz <= 512}
        yr_waits = {sz: pltpu.make_async_remote_copy(
            returns.at[pl.ds(0, sz), :], returns.at[pl.ds(0, sz), :],
            sems.at[Y_SEND], sems.at[Y_RECV],
            device_id=my, device_id_type=LOGICAL) for sz in SZ11}

        def bit_waits(n, descs, sizes, recv):
            for sz in sizes:
                @pl.when((n & sz) != 0)
                def _(sz=sz):
                    if recv:
                        descs[sz].wait_recv()
                    else:
                        descs[sz].wait_send()

        for s in range(E_LOC):
            base = s * CAP_PAD
            ks = kept_slot[s]

            with jax.named_scope(f"slot{s}"):
                bit_waits(ks, {sz: xr_waits[(s, sz)] for sz in SZ8},
                          SZ8, recv=True)

                meta_vmem[base:base + CAP_PAD, :] = (
                    xbuf[base:base + CAP_PAD, D:D + 1].astype(i32))
                mcopy = pltpu.make_async_copy(
                    meta_vmem.at[pl.ds(base, CAP_PAD), :],
                    meta_smem.at[pl.ds(base, CAP_PAD), :], local_sem)
                mcopy.start()

                a = xbuf[base:base + CAP_PAD, 0:D]
                ybuf[base:base + CAP_PAD, :] = jnp.dot(
                    a, w_ref[s], preferred_element_type=f32)
                mcopy.wait()

                def comb(rr, c, base=base, ks=ks):
                    @pl.when(rr < ks)
                    def _():
                        m = meta_smem[base + rr, 0]
                        pltpu.make_async_remote_copy(
                            ybuf.at[pl.ds(base + rr, 1), :],
                            returns.at[pl.ds(m % T, 1), :],
                            sems.at[Y_SEND], sems.at[Y_RECV],
                            device_id=m // T, device_id_type=LOGICAL,
                        ).start()
                    return c
                lax.fori_loop(0, CAP, comb, 0, unroll=8)

        with jax.named_scope("drain"):
            bit_waits(kept_mine, xs_waits, SZ11, recv=False)
            bit_waits(kept_owner, ys_waits,
                      tuple(sz for sz in SZ11 if sz <= 512), recv=False)
            bit_waits(kept_mine, yr_waits, SZ11, recv=True)

        with jax.named_scope("out_cast"):
            out_ref[...] = jnp.where(keep_vec, returns[...], 0.0)

        with jax.named_scope("exit_barrier"):
            def esig(d, c):
                @pl.when(d != my)
                def _():
                    pl.semaphore_signal(exit_sem, 1, device_id=d,
                                        device_id_type=LOGICAL)
                return c
            lax.fori_loop(0, N_DEV, esig, 0)
            pl.semaphore_wait(exit_sem, N_DEV - 1)

    return pl.pallas_call(
        body,
        out_shape=jax.ShapeDtypeStruct((T, H), f32),
        in_specs=[pl.BlockSpec(memory_space=pltpu.VMEM)] * 3,
        out_specs=pl.BlockSpec(memory_space=pltpu.VMEM),
        scratch_shapes=[
            pltpu.VMEM((1, E), i32),
            pltpu.VMEM((N_DEV, E), i32),
            pltpu.VMEM((3, E), i32),
            pltpu.VMEM((T, 4), i32),
            pltpu.VMEM((T, DX), f32),
            pltpu.VMEM((E_LOC * CAP_PAD, DX), f32),
            pltpu.VMEM((E_LOC * CAP_PAD, H), f32),
            pltpu.VMEM((T, H), f32),
            pltpu.VMEM((E_LOC * CAP_PAD, 1), i32),
            pltpu.SMEM((3, E), i32),
            pltpu.SMEM((T, 4), i32),
            pltpu.SMEM((E_LOC * CAP_PAD, 1), i32),
            pltpu.SemaphoreType.DMA((9,)),
            pltpu.SemaphoreType.DMA,
            pltpu.SemaphoreType.REGULAR,
        ],
        compiler_params=pltpu.CompilerParams(collective_id=0),
    )(x, route_idx, expert_W)
